# Initial kernel scaffold; baseline (speedup 1.0000x reference)
import functools

import jax
import jax.numpy as jnp
from jax import lax
from jax.experimental import pallas as pl
from jax.experimental.pallas import tpu as pltpu
from jax.experimental.pallas import tpu_sc as plsc

N_NODES = 10000
N_EDGES = 320000
C = 128

NC = 2
NS = 16
NW = NC * NS

EPT = N_EDGES // NW
CHUNK = 625
NCHUNK = EPT // CHUNK
IDXW = 125
IDXR = CHUNK // IDXW
VPT = N_NODES // NS


def _sc_segsum_body(x_hbm, vid_hbm, z128_hbm, z16_hbm, ones_hbm,
                    part_hbm, cnt_hbm,
                    acc, cacc, xbuf, idxbuf, onesbuf):
    cid = lax.axis_index("c")
    sid = lax.axis_index("s")
    wid = cid * NS + sid
    pltpu.sync_copy(z128_hbm, acc.at[pl.ds(VPT * sid, VPT)])
    pltpu.sync_copy(z16_hbm, cacc.at[pl.ds(VPT * sid, VPT)])
    pltpu.sync_copy(ones_hbm, onesbuf)
    plsc.subcore_barrier()
    for c in range(NCHUNK):
        e0 = wid * EPT + CHUNK * c
        r0 = e0 // IDXW
        pltpu.sync_copy(x_hbm.at[pl.ds(e0, CHUNK)], xbuf)
        pltpu.sync_copy(vid_hbm.at[pl.ds(r0, IDXR)], idxbuf)
        for j in range(IDXR):
            pltpu.sync_copy(xbuf.at[pl.ds(IDXW * j, IDXW)],
                            acc.at[idxbuf.at[j]], add=True)
            pltpu.sync_copy(onesbuf.at[pl.ds(IDXW * j, IDXW)],
                            cacc.at[idxbuf.at[j]], add=True)
    plsc.subcore_barrier()
    row0 = cid * N_NODES + VPT * sid
    pltpu.sync_copy(acc.at[pl.ds(VPT * sid, VPT)],
                    part_hbm.at[pl.ds(row0, VPT)])
    pltpu.sync_copy(cacc.at[pl.ds(VPT * sid, VPT)],
                    cnt_hbm.at[pl.ds(row0, VPT)])


def _sc_segsum(x, vid2, z128, z16, ones16):
    mesh = plsc.VectorSubcoreMesh(core_axis_name="c", subcore_axis_name="s")
    return pl.kernel(
        _sc_segsum_body,
        out_type=(
            jax.ShapeDtypeStruct((NC * N_NODES, C), jnp.float32),
            jax.ShapeDtypeStruct((NC * N_NODES, 16), jnp.float32),
        ),
        mesh=mesh,
        scratch_types=[
            pltpu.VMEM_SHARED((N_NODES, C), jnp.float32),
            pltpu.VMEM_SHARED((N_NODES, 16), jnp.float32),
            pltpu.VMEM((CHUNK, C), jnp.float32),
            pltpu.VMEM((IDXR, IDXW), jnp.int32),
            pltpu.VMEM((CHUNK, 16), jnp.float32),
        ],
    )(x, vid2, z128, z16, ones16)


def _sc_gather_body(zv_hbm, vid_hbm, g_hbm, idxbuf, gbuf, sem):
    cid = lax.axis_index("c")
    sid = lax.axis_index("s")
    wid = cid * NS + sid
    for c in range(NCHUNK):
        e0 = wid * EPT + CHUNK * c
        r0 = e0 // IDXW
        pltpu.sync_copy(vid_hbm.at[pl.ds(r0, IDXR)], idxbuf)
        for j in range(IDXR):
            pltpu.async_copy(zv_hbm.at[idxbuf.at[j]],
                             gbuf.at[pl.ds(IDXW * j, IDXW)], sem).wait()
        pltpu.sync_copy(gbuf, g_hbm.at[pl.ds(e0, CHUNK)])


def _sc_gather(zv, vid2):
    mesh = plsc.VectorSubcoreMesh(core_axis_name="c", subcore_axis_name="s")
    return pl.kernel(
        _sc_gather_body,
        out_type=jax.ShapeDtypeStruct((N_EDGES, C), jnp.float32),
        mesh=mesh,
        scratch_types=[
            pltpu.VMEM((IDXR, IDXW), jnp.int32),
            pltpu.VMEM((CHUNK, C), jnp.float32),
            pltpu.SemaphoreType.DMA,
        ],
    )(zv, vid2)


def _tc_vertex_body(p0, p1, c0, c1, w1, wv, b1, bv, zv):
    s = p0[...] + p1[...]
    cnt = c0[:, 0:1] + c1[:, 0:1]
    w1v = jnp.dot(w1[...], wv[...], preferred_element_type=jnp.float32)
    b1v = jnp.dot(b1[...], wv[...], preferred_element_type=jnp.float32)
    zv[...] = (jnp.dot(s, w1v, preferred_element_type=jnp.float32)
               + cnt * b1v + bv[...])


def _tc_vertex(part, cnt, W1, b1, Wv, bv):
    vb = 1000
    grid = N_NODES // vb
    return pl.pallas_call(
        _tc_vertex_body,
        grid=(grid,),
        in_specs=[
            pl.BlockSpec((vb, C), lambda i: (i, 0)),
            pl.BlockSpec((vb, C), lambda i: (i + grid, 0)),
            pl.BlockSpec((vb, 16), lambda i: (i, 0)),
            pl.BlockSpec((vb, 16), lambda i: (i + grid, 0)),
            pl.BlockSpec((C, C), lambda i: (0, 0)),
            pl.BlockSpec((C, C), lambda i: (0, 0)),
            pl.BlockSpec((1, C), lambda i: (0, 0)),
            pl.BlockSpec((1, C), lambda i: (0, 0)),
        ],
        out_specs=pl.BlockSpec((vb, C), lambda i: (i, 0)),
        out_shape=jax.ShapeDtypeStruct((N_NODES, C), jnp.float32),
    )(part, part, cnt, cnt, W1, Wv, b1.reshape(1, C), bv.reshape(1, C))


def _tc_edge_body(x, g, we, be, out):
    out[...] = (jnp.dot(x[...], we[...], preferred_element_type=jnp.float32)
                + be[...] + g[...])


def _tc_edge(x, g, We, be):
    eb = 2000
    grid = N_EDGES // eb
    return pl.pallas_call(
        _tc_edge_body,
        grid=(grid,),
        in_specs=[
            pl.BlockSpec((eb, C), lambda i: (i, 0)),
            pl.BlockSpec((eb, C), lambda i: (i, 0)),
            pl.BlockSpec((C, C), lambda i: (0, 0)),
            pl.BlockSpec((1, C), lambda i: (0, 0)),
        ],
        out_specs=pl.BlockSpec((eb, C), lambda i: (i, 0)),
        out_shape=jax.ShapeDtypeStruct((N_EDGES, C), jnp.float32),
    )(x, g, We, be.reshape(1, C))


def kernel(x, vertex_id, W1, b1, We, be, Wv, bv):
    vid2 = vertex_id.reshape(N_EDGES // IDXW, IDXW)
    z128 = jnp.zeros((VPT, C), jnp.float32)
    z16 = jnp.zeros((VPT, 16), jnp.float32)
    ones16 = jnp.ones((CHUNK, 16), jnp.float32)
    part, cnt = _sc_segsum(x, vid2, z128, z16, ones16)
    zv = _tc_vertex(part, cnt, W1, b1, Wv, bv)
    g = _sc_gather(zv, vid2)
    return _tc_edge(x, g, We, be)


# speedup vs baseline: 2.3085x; 2.3085x over previous
import functools

import jax
import jax.numpy as jnp
from jax import lax
from jax.experimental import pallas as pl
from jax.experimental.pallas import tpu as pltpu
from jax.experimental.pallas import tpu_sc as plsc

N_NODES = 10000
N_EDGES = 320000
C = 128

NC = 2
NS = 16
NW = NC * NS

CHUNK = 625
IDXW = 125
IDXR = CHUNK // IDXW
VPT = N_NODES // NS
CH = C // NC

EPT1 = N_EDGES // NS
NCHUNK1 = EPT1 // CHUNK


def _sc_segsum_body(x_hbm, vid_hbm, z64_hbm, z16_hbm, ones_hbm,
                    part_hbm, cnt_hbm,
                    acc, cacc, xbuf, idxbuf, onesbuf):
    cid = lax.axis_index("c")
    sid = lax.axis_index("s")
    col0 = cid * CH
    pltpu.sync_copy(z64_hbm, acc.at[pl.ds(VPT * sid, VPT)])

    @pl.when(cid == 0)
    def _():
        pltpu.sync_copy(z16_hbm, cacc.at[pl.ds(VPT * sid, VPT)])
        pltpu.sync_copy(ones_hbm, onesbuf)

    plsc.subcore_barrier()
    for c in range(NCHUNK1):
        e0 = sid * EPT1 + CHUNK * c
        r0 = e0 // IDXW
        pltpu.sync_copy(x_hbm.at[pl.ds(e0, CHUNK), pl.ds(col0, CH)], xbuf)
        pltpu.sync_copy(vid_hbm.at[pl.ds(r0, IDXR)], idxbuf)
        for j in range(IDXR):
            pltpu.sync_copy(xbuf.at[pl.ds(IDXW * j, IDXW)],
                            acc.at[idxbuf.at[j]], add=True)

        @pl.when(cid == 0)
        def _():
            for j in range(IDXR):
                pltpu.sync_copy(onesbuf.at[pl.ds(IDXW * j, IDXW)],
                                cacc.at[idxbuf.at[j]], add=True)

    plsc.subcore_barrier()
    pltpu.sync_copy(acc.at[pl.ds(VPT * sid, VPT)],
                    part_hbm.at[pl.ds(VPT * sid, VPT), pl.ds(col0, CH)])

    @pl.when(cid == 0)
    def _():
        pltpu.sync_copy(cacc.at[pl.ds(VPT * sid, VPT)],
                        cnt_hbm.at[pl.ds(VPT * sid, VPT)])


def _sc_segsum(x, vid2, z64, z16, ones16):
    mesh = plsc.VectorSubcoreMesh(core_axis_name="c", subcore_axis_name="s")
    return pl.kernel(
        _sc_segsum_body,
        compiler_params=pltpu.CompilerParams(use_tc_tiling_on_sc=False),
        out_type=(
            jax.ShapeDtypeStruct((N_NODES, C), jnp.float32),
            jax.ShapeDtypeStruct((N_NODES, 16), jnp.float32),
        ),
        mesh=mesh,
        scratch_types=[
            pltpu.VMEM_SHARED((N_NODES, CH), jnp.float32),
            pltpu.VMEM_SHARED((N_NODES, 16), jnp.float32),
            pltpu.VMEM((CHUNK, CH), jnp.float32),
            pltpu.VMEM((IDXR, IDXW), jnp.int32),
            pltpu.VMEM((CHUNK, 16), jnp.float32),
        ],
    )(x, vid2, z64, z16, ones16)


EPT3 = N_EDGES // NW
NCHUNK3 = EPT3 // CHUNK


def _sc_gather_body(zv_hbm, vid_hbm, g_hbm, idxbuf, gbuf, sem):
    cid = lax.axis_index("c")
    sid = lax.axis_index("s")
    wid = cid * NS + sid
    for c in range(NCHUNK3):
        e0 = wid * EPT3 + CHUNK * c
        r0 = e0 // IDXW
        pltpu.sync_copy(vid_hbm.at[pl.ds(r0, IDXR)], idxbuf)
        for j in range(IDXR):
            pltpu.async_copy(zv_hbm.at[idxbuf.at[j]],
                             gbuf.at[pl.ds(IDXW * j, IDXW)], sem).wait()
        pltpu.sync_copy(gbuf, g_hbm.at[pl.ds(e0, CHUNK)])


def _sc_gather(zv, vid2):
    mesh = plsc.VectorSubcoreMesh(core_axis_name="c", subcore_axis_name="s")
    return pl.kernel(
        _sc_gather_body,
        compiler_params=pltpu.CompilerParams(use_tc_tiling_on_sc=False),
        out_type=jax.ShapeDtypeStruct((N_EDGES, C), jnp.float32),
        mesh=mesh,
        scratch_types=[
            pltpu.VMEM((IDXR, IDXW), jnp.int32),
            pltpu.VMEM((CHUNK, C), jnp.float32),
            pltpu.SemaphoreType.DMA,
        ],
    )(zv, vid2)


def _tc_vertex_body(p, cn, w1, wv, b1, bv, zv):
    s = p[...]
    cnt = cn[:, 0:1]
    w1v = jnp.dot(w1[...], wv[...], preferred_element_type=jnp.float32)
    b1v = jnp.dot(b1[...], wv[...], preferred_element_type=jnp.float32)
    zv[...] = (jnp.dot(s, w1v, preferred_element_type=jnp.float32)
               + cnt * b1v + bv[...])


def _tc_vertex(part, cnt, W1, b1, Wv, bv):
    vb = 1000
    grid = N_NODES // vb
    return pl.pallas_call(
        _tc_vertex_body,
        grid=(grid,),
        in_specs=[
            pl.BlockSpec((vb, C), lambda i: (i, 0)),
            pl.BlockSpec((vb, 16), lambda i: (i, 0)),
            pl.BlockSpec((C, C), lambda i: (0, 0)),
            pl.BlockSpec((C, C), lambda i: (0, 0)),
            pl.BlockSpec((1, C), lambda i: (0, 0)),
            pl.BlockSpec((1, C), lambda i: (0, 0)),
        ],
        out_specs=pl.BlockSpec((vb, C), lambda i: (i, 0)),
        out_shape=jax.ShapeDtypeStruct((N_NODES, C), jnp.float32),
    )(part, cnt, W1, Wv, b1.reshape(1, C), bv.reshape(1, C))


def _tc_edge_body(x, g, we, be, out):
    out[...] = (jnp.dot(x[...], we[...], preferred_element_type=jnp.float32)
                + be[...] + g[...])


def _tc_edge(x, g, We, be):
    eb = 2000
    grid = N_EDGES // eb
    return pl.pallas_call(
        _tc_edge_body,
        grid=(grid,),
        in_specs=[
            pl.BlockSpec((eb, C), lambda i: (i, 0)),
            pl.BlockSpec((eb, C), lambda i: (i, 0)),
            pl.BlockSpec((C, C), lambda i: (0, 0)),
            pl.BlockSpec((1, C), lambda i: (0, 0)),
        ],
        out_specs=pl.BlockSpec((eb, C), lambda i: (i, 0)),
        out_shape=jax.ShapeDtypeStruct((N_EDGES, C), jnp.float32),
    )(x, g, We, be.reshape(1, C))


def kernel(x, vertex_id, W1, b1, We, be, Wv, bv):
    vid2 = vertex_id.reshape(N_EDGES // IDXW, IDXW)
    z64 = jnp.zeros((VPT, CH), jnp.float32)
    z16 = jnp.zeros((VPT, 16), jnp.float32)
    ones16 = jnp.ones((CHUNK, 16), jnp.float32)
    part, cnt = _sc_segsum(x, vid2, z64, z16, ones16)
    zv = _tc_vertex(part, cnt, W1, b1, Wv, bv)
    g = _sc_gather(zv, vid2)
    return _tc_edge(x, g, We, be)
